# Initial kernel scaffold; baseline (speedup 1.0000x reference)
import jax
import jax.numpy as jnp
from jax.experimental import pallas as pl


def kernel(x, edge_index, edge_attr, batch, en1_w1, en1_b1, en1_w2, en1_b2, root1, bias1, en2_w1, en2_b1, en2_w2, en2_b2, root2, bias2, en3_w1, en3_b1, en3_w2, en3_b2, root3, bias3, cls_w1, cls_b1, cls_w2, cls_b2):
    raise NotImplementedError("write your pallas kernel here")


# speedup vs baseline: 3.0589x; 3.0589x over previous
import jax
import jax.numpy as jnp
from jax import lax
from jax.experimental import pallas as pl
from jax.experimental.pallas import tpu as pltpu
from jax.experimental.pallas import tpu_sc as plsc

N = 10000
E = 160000
F = 16
ED = 4
NG = 8
NC = 2
NS = 16
NW = NC * NS
CH = 128
NCH = 40
EPW = NCH * CH
EP = NW * EPW
ACC_N = 10240
RPT = ACC_N // NS
EB = 2048

_mesh = plsc.VectorSubcoreMesh(
    core_axis_name="c", subcore_axis_name="s", num_cores=NC, num_subcores=NS)
_sc_params = pltpu.CompilerParams(use_tc_tiling_on_sc=False)


def _wid():
    return lax.axis_index("s") * NC + lax.axis_index("c")


def _gather_body(x_hbm, idx_hbm, out_hbm, idx_v, rows_v, sem):
    w = _wid()
    pltpu.sync_copy(idx_hbm.at[w], idx_v)

    def chunk(ci, carry):
        pltpu.async_copy(x_hbm.at[idx_v.at[ci]], rows_v.at[ci], sem).wait()
        return carry

    lax.fori_loop(0, NCH, chunk, 0)
    pltpu.sync_copy(rows_v, out_hbm.at[w])


_gather = pl.kernel(
    _gather_body,
    out_type=jax.ShapeDtypeStruct((NW, NCH, CH, F), jnp.float32),
    mesh=_mesh,
    scratch_types=[
        pltpu.VMEM((NCH, CH), jnp.int32),
        pltpu.VMEM((NCH, CH, F), jnp.float32),
        pltpu.SemaphoreType.DMA,
    ],
    compiler_params=_sc_params,
)


def _scatter_body(m_hbm, idx_hbm, zeros_hbm, out_hbm, idx_v, rows_v, acc):
    cid = lax.axis_index("c")
    sid = lax.axis_index("s")
    w = _wid()
    pltpu.sync_copy(zeros_hbm.at[pl.ds(sid * RPT, RPT)],
                    acc.at[pl.ds(sid * RPT, RPT)])
    pltpu.sync_copy(idx_hbm.at[w], idx_v)
    pltpu.sync_copy(m_hbm.at[w], rows_v)
    plsc.subcore_barrier()

    def chunk(ci, carry):
        pltpu.sync_copy(rows_v.at[ci], acc.at[idx_v.at[ci]], add=True)
        return carry

    lax.fori_loop(0, NCH, chunk, 0)
    plsc.subcore_barrier()
    pltpu.sync_copy(acc.at[pl.ds(sid * RPT, RPT)],
                    out_hbm.at[cid, pl.ds(sid * RPT, RPT)])


_scatter = pl.kernel(
    _scatter_body,
    out_type=jax.ShapeDtypeStruct((NC, ACC_N, F), jnp.float32),
    mesh=_mesh,
    scratch_types=[
        pltpu.VMEM((NCH, CH), jnp.int32),
        pltpu.VMEM((NCH, CH, F), jnp.float32),
        pltpu.VMEM_SHARED((ACC_N, F), jnp.float32),
    ],
    compiler_params=_sc_params,
)


def _count_body(idx_hbm, zeros_hbm, onepat_hbm, out_hbm, idx_v, obuf, acc):
    cid = lax.axis_index("c")
    sid = lax.axis_index("s")
    w = _wid()
    pltpu.sync_copy(zeros_hbm.at[pl.ds(sid * RPT, RPT)],
                    acc.at[pl.ds(sid * RPT, RPT)])
    pltpu.sync_copy(idx_hbm.at[w], idx_v)
    pltpu.sync_copy(onepat_hbm, obuf)
    plsc.subcore_barrier()

    def chunk(ci, carry):
        pltpu.sync_copy(obuf, acc.at[idx_v.at[ci]], add=True)
        return carry

    lax.fori_loop(0, NCH, chunk, 0)
    plsc.subcore_barrier()
    pltpu.sync_copy(acc.at[pl.ds(sid * RPT, RPT)],
                    out_hbm.at[cid, pl.ds(sid * RPT, RPT)])


_count = pl.kernel(
    _count_body,
    out_type=jax.ShapeDtypeStruct((NC, ACC_N, F), jnp.float32),
    mesh=_mesh,
    scratch_types=[
        pltpu.VMEM((NCH, CH), jnp.int32),
        pltpu.VMEM((CH, F), jnp.float32),
        pltpu.VMEM_SHARED((ACC_N, F), jnp.float32),
    ],
    compiler_params=_sc_params,
)


def _edge_body(xj_ref, ea_ref, w1_ref, b1_ref, w2_ref, b2_ref, r_ref, s_ref,
               m_ref):
    i = pl.program_id(0)
    row = lax.broadcasted_iota(jnp.int32, (EB, F), 0) + i * EB
    xj = jnp.where(row < E, xj_ref[...], 0.0)
    h = jnp.maximum(ea_ref[...] @ w1_ref[...] + b1_ref[...], 0.0)
    we = h @ w2_ref[...] + b2_ref[...]
    x2 = xj @ r_ref[...]
    m_ref[...] = (x2 * we) @ s_ref[...]


def _edge(xj, ea, w1, b1, w2, b2, rm, sm):
    return pl.pallas_call(
        _edge_body,
        grid=(EP // EB,),
        in_specs=[
            pl.BlockSpec((EB, F), lambda i: (i, 0)),
            pl.BlockSpec((EB, ED), lambda i: (i, 0)),
            pl.BlockSpec((ED, 32), lambda i: (0, 0)),
            pl.BlockSpec((1, 32), lambda i: (0, 0)),
            pl.BlockSpec((32, 256), lambda i: (0, 0)),
            pl.BlockSpec((1, 256), lambda i: (0, 0)),
            pl.BlockSpec((F, 256), lambda i: (0, 0)),
            pl.BlockSpec((256, F), lambda i: (0, 0)),
        ],
        out_specs=pl.BlockSpec((EB, F), lambda i: (i, 0)),
        out_shape=jax.ShapeDtypeStruct((EP, F), jnp.float32),
    )(xj, ea, w1, b1, w2, b2, rm, sm)


def _fin_body(x_ref, parts_ref, cnt_ref, root_ref, bias_ref, o_ref):
    s = parts_ref[0, :N, :] + parts_ref[1, :N, :]
    c0 = cnt_ref[0, :N, 0:1] + cnt_ref[1, :N, 0:1]
    agg = s / jnp.maximum(c0, 1.0)
    o_ref[...] = jnp.maximum(
        agg + x_ref[...] @ root_ref[...] + bias_ref[...], 0.0)


def _finalize(x, parts, cnt, root, bias):
    return pl.pallas_call(
        _fin_body,
        out_shape=jax.ShapeDtypeStruct((N, F), jnp.float32),
    )(x, parts, cnt, root, bias)


def _pool_body(x_ref, b_ref, w1_ref, b1_ref, w2_ref, b2_ref, o_ref):
    xv = x_ref[...]
    oh = (b_ref[...] == lax.broadcasted_iota(jnp.int32, (1, NG), 1)
          ).astype(jnp.float32)
    ps = lax.dot_general(oh, xv, (((0,), (0,)), ((), ())))
    pc = lax.dot_general(oh, jnp.ones((N, 1), jnp.float32),
                         (((0,), (0,)), ((), ())))
    p = ps / jnp.maximum(pc, 1.0)
    h = jnp.maximum(p @ w1_ref[...] + b1_ref[...], 0.0)
    o_ref[...] = h @ w2_ref[...] + b2_ref[...]


def _pool(x, batch2d, w1, b1, w2, b2):
    return pl.pallas_call(
        _pool_body,
        out_shape=jax.ShapeDtypeStruct((NG, 2), jnp.float32),
    )(x, batch2d, w1, b1, w2, b2)


def kernel(x, edge_index, edge_attr, batch,
           en1_w1, en1_b1, en1_w2, en1_b2, root1, bias1,
           en2_w1, en2_b1, en2_w2, en2_b2, root2, bias2,
           en3_w1, en3_b1, en3_w2, en3_b2, root3, bias3,
           cls_w1, cls_b1, cls_w2, cls_b2):
    pad = EP - E
    src = jnp.concatenate(
        [edge_index[0], jnp.zeros((pad,), jnp.int32)]).reshape(NW, NCH, CH)
    dst = jnp.concatenate(
        [edge_index[1], jnp.full((pad,), N, jnp.int32)]).reshape(NW, NCH, CH)
    ea = jnp.concatenate(
        [edge_attr, jnp.zeros((pad, ED), jnp.float32)])
    zeros_acc = jnp.zeros((ACC_N, F), jnp.float32)
    onepat = jnp.zeros((CH, F), jnp.float32).at[:, 0].set(1.0)
    rm = jnp.repeat(jnp.eye(F, dtype=jnp.float32), F, axis=1)
    sm = jnp.tile(jnp.eye(F, dtype=jnp.float32), (F, 1))

    cnt = _count(dst, zeros_acc, onepat)

    xc = x
    layers = [
        (en1_w1, en1_b1, en1_w2, en1_b2, root1, bias1),
        (en2_w1, en2_b1, en2_w2, en2_b2, root2, bias2),
        (en3_w1, en3_b1, en3_w2, en3_b2, root3, bias3),
    ]
    for w1, b1, w2, b2, root, bias in layers:
        xj = _gather(xc, src).reshape(EP, F)
        m = _edge(xj, ea, w1, b1.reshape(1, 32), w2, b2.reshape(1, 256),
                  rm, sm)
        parts = _scatter(m.reshape(NW, NCH, CH, F), dst, zeros_acc)
        xc = _finalize(xc, parts, cnt, root, bias.reshape(1, F))

    return _pool(xc, batch.reshape(N, 1), cls_w1, cls_b1.reshape(1, NG),
                 cls_w2, cls_b2.reshape(1, 2))


# speedup vs baseline: 3.1349x; 1.0248x over previous
import jax
import jax.numpy as jnp
from jax import lax
from jax.experimental import pallas as pl
from jax.experimental.pallas import tpu as pltpu
from jax.experimental.pallas import tpu_sc as plsc

N = 10000
E = 160000
F = 16
ED = 4
NG = 8
NC = 2
NS = 16
NW = NC * NS
CH = 128
NCH = 40
EPW = NCH * CH
EP = NW * EPW
ACC_N = 10240
RPT = ACC_N // NS
EB = 2048

_mesh = plsc.VectorSubcoreMesh(
    core_axis_name="c", subcore_axis_name="s", num_cores=NC, num_subcores=NS)
_sc_params = pltpu.CompilerParams(use_tc_tiling_on_sc=False)


def _wid():
    return lax.axis_index("s") * NC + lax.axis_index("c")


def _gather_body(x_hbm, idx_hbm, out_hbm, idx_v, rows_v, sem):
    w = _wid()
    pltpu.sync_copy(idx_hbm.at[w], idx_v)

    def fire(ci, carry):
        pltpu.async_copy(x_hbm.at[idx_v.at[ci]], rows_v.at[ci], sem)
        return carry

    lax.fori_loop(0, NCH, fire, 0)
    pltpu.make_async_copy(out_hbm.at[w], rows_v, sem).wait()
    pltpu.sync_copy(rows_v, out_hbm.at[w])


_gather = pl.kernel(
    _gather_body,
    out_type=jax.ShapeDtypeStruct((NW, NCH, CH, F), jnp.float32),
    mesh=_mesh,
    scratch_types=[
        pltpu.VMEM((NCH, CH), jnp.int32),
        pltpu.VMEM((NCH, CH, F), jnp.float32),
        pltpu.SemaphoreType.DMA,
    ],
    compiler_params=_sc_params,
)


def _scatter_body(m_hbm, idx_hbm, zeros_hbm, out_hbm, idx_v, rows_v, acc,
                  sem):
    cid = lax.axis_index("c")
    sid = lax.axis_index("s")
    w = _wid()
    pltpu.sync_copy(zeros_hbm.at[pl.ds(sid * RPT, RPT)],
                    acc.at[pl.ds(sid * RPT, RPT)])
    pltpu.sync_copy(idx_hbm.at[w], idx_v)
    pltpu.sync_copy(m_hbm.at[w], rows_v)
    plsc.subcore_barrier()

    def fire(ci, carry):
        pltpu.async_copy(rows_v.at[ci], acc.at[idx_v.at[ci]], sem, add=True)
        return carry

    lax.fori_loop(0, NCH, fire, 0)
    pltpu.make_async_copy(m_hbm.at[w], rows_v, sem).wait()
    plsc.subcore_barrier()
    pltpu.sync_copy(acc.at[pl.ds(sid * RPT, RPT)],
                    out_hbm.at[cid, pl.ds(sid * RPT, RPT)])


_scatter = pl.kernel(
    _scatter_body,
    out_type=jax.ShapeDtypeStruct((NC, ACC_N, F), jnp.float32),
    mesh=_mesh,
    scratch_types=[
        pltpu.VMEM((NCH, CH), jnp.int32),
        pltpu.VMEM((NCH, CH, F), jnp.float32),
        pltpu.VMEM_SHARED((ACC_N, F), jnp.float32),
        pltpu.SemaphoreType.DMA,
    ],
    compiler_params=_sc_params,
)


def _count_body(idx_hbm, zeros_hbm, onepat_hbm, out_hbm, idx_v, obuf, acc,
                sem):
    cid = lax.axis_index("c")
    sid = lax.axis_index("s")
    w = _wid()
    pltpu.sync_copy(zeros_hbm.at[pl.ds(sid * RPT, RPT)],
                    acc.at[pl.ds(sid * RPT, RPT)])
    pltpu.sync_copy(idx_hbm.at[w], idx_v)
    pltpu.sync_copy(onepat_hbm, obuf)
    plsc.subcore_barrier()

    def fire(ci, carry):
        pltpu.async_copy(obuf, acc.at[idx_v.at[ci]], sem, add=True)
        return carry

    lax.fori_loop(0, NCH, fire, 0)

    def drain(ci, carry):
        pltpu.make_async_copy(onepat_hbm, obuf, sem).wait()
        return carry

    lax.fori_loop(0, NCH, drain, 0)
    plsc.subcore_barrier()
    pltpu.sync_copy(acc.at[pl.ds(sid * RPT, RPT)],
                    out_hbm.at[cid, pl.ds(sid * RPT, RPT)])


_count = pl.kernel(
    _count_body,
    out_type=jax.ShapeDtypeStruct((NC, ACC_N, F), jnp.float32),
    mesh=_mesh,
    scratch_types=[
        pltpu.VMEM((NCH, CH), jnp.int32),
        pltpu.VMEM((CH, F), jnp.float32),
        pltpu.VMEM_SHARED((ACC_N, F), jnp.float32),
        pltpu.SemaphoreType.DMA,
    ],
    compiler_params=_sc_params,
)


def _edge_body(xj_ref, ea_ref, w1_ref, b1_ref, w2_ref, b2_ref, r_ref, s_ref,
               m_ref):
    i = pl.program_id(0)
    row = lax.broadcasted_iota(jnp.int32, (EB, F), 0) + i * EB
    xj = jnp.where(row < E, xj_ref[...], 0.0)
    h = jnp.maximum(ea_ref[...] @ w1_ref[...] + b1_ref[...], 0.0)
    we = h @ w2_ref[...] + b2_ref[...]
    x2 = xj @ r_ref[...]
    m_ref[...] = (x2 * we) @ s_ref[...]


def _edge(xj, ea, w1, b1, w2, b2, rm, sm):
    return pl.pallas_call(
        _edge_body,
        grid=(EP // EB,),
        in_specs=[
            pl.BlockSpec((EB, F), lambda i: (i, 0)),
            pl.BlockSpec((EB, ED), lambda i: (i, 0)),
            pl.BlockSpec((ED, 32), lambda i: (0, 0)),
            pl.BlockSpec((1, 32), lambda i: (0, 0)),
            pl.BlockSpec((32, 256), lambda i: (0, 0)),
            pl.BlockSpec((1, 256), lambda i: (0, 0)),
            pl.BlockSpec((F, 256), lambda i: (0, 0)),
            pl.BlockSpec((256, F), lambda i: (0, 0)),
        ],
        out_specs=pl.BlockSpec((EB, F), lambda i: (i, 0)),
        out_shape=jax.ShapeDtypeStruct((EP, F), jnp.float32),
    )(xj, ea, w1, b1, w2, b2, rm, sm)


def _fin_body(x_ref, parts_ref, cnt_ref, root_ref, bias_ref, o_ref):
    s = parts_ref[0, :N, :] + parts_ref[1, :N, :]
    c0 = cnt_ref[0, :N, 0:1] + cnt_ref[1, :N, 0:1]
    agg = s / jnp.maximum(c0, 1.0)
    o_ref[...] = jnp.maximum(
        agg + x_ref[...] @ root_ref[...] + bias_ref[...], 0.0)


def _finalize(x, parts, cnt, root, bias):
    return pl.pallas_call(
        _fin_body,
        out_shape=jax.ShapeDtypeStruct((N, F), jnp.float32),
    )(x, parts, cnt, root, bias)


def _pool_body(x_ref, b_ref, w1_ref, b1_ref, w2_ref, b2_ref, o_ref):
    xv = x_ref[...]
    oh = (b_ref[...] == lax.broadcasted_iota(jnp.int32, (1, NG), 1)
          ).astype(jnp.float32)
    ps = lax.dot_general(oh, xv, (((0,), (0,)), ((), ())))
    pc = lax.dot_general(oh, jnp.ones((N, 1), jnp.float32),
                         (((0,), (0,)), ((), ())))
    p = ps / jnp.maximum(pc, 1.0)
    h = jnp.maximum(p @ w1_ref[...] + b1_ref[...], 0.0)
    o_ref[...] = h @ w2_ref[...] + b2_ref[...]


def _pool(x, batch2d, w1, b1, w2, b2):
    return pl.pallas_call(
        _pool_body,
        out_shape=jax.ShapeDtypeStruct((NG, 2), jnp.float32),
    )(x, batch2d, w1, b1, w2, b2)


def kernel(x, edge_index, edge_attr, batch,
           en1_w1, en1_b1, en1_w2, en1_b2, root1, bias1,
           en2_w1, en2_b1, en2_w2, en2_b2, root2, bias2,
           en3_w1, en3_b1, en3_w2, en3_b2, root3, bias3,
           cls_w1, cls_b1, cls_w2, cls_b2):
    pad = EP - E
    src = jnp.concatenate(
        [edge_index[0], jnp.zeros((pad,), jnp.int32)]).reshape(NW, NCH, CH)
    dst = jnp.concatenate(
        [edge_index[1], jnp.full((pad,), N, jnp.int32)]).reshape(NW, NCH, CH)
    ea = jnp.concatenate(
        [edge_attr, jnp.zeros((pad, ED), jnp.float32)])
    zeros_acc = jnp.zeros((ACC_N, F), jnp.float32)
    onepat = jnp.zeros((CH, F), jnp.float32).at[:, 0].set(1.0)
    rm = jnp.repeat(jnp.eye(F, dtype=jnp.float32), F, axis=1)
    sm = jnp.tile(jnp.eye(F, dtype=jnp.float32), (F, 1))

    cnt = _count(dst, zeros_acc, onepat)

    xc = x
    layers = [
        (en1_w1, en1_b1, en1_w2, en1_b2, root1, bias1),
        (en2_w1, en2_b1, en2_w2, en2_b2, root2, bias2),
        (en3_w1, en3_b1, en3_w2, en3_b2, root3, bias3),
    ]
    for w1, b1, w2, b2, root, bias in layers:
        xj = _gather(xc, src).reshape(EP, F)
        m = _edge(xj, ea, w1, b1.reshape(1, 32), w2, b2.reshape(1, 256),
                  rm, sm)
        parts = _scatter(m.reshape(NW, NCH, CH, F), dst, zeros_acc)
        xc = _finalize(xc, parts, cnt, root, bias.reshape(1, F))

    return _pool(xc, batch.reshape(N, 1), cls_w1, cls_b1.reshape(1, NG),
                 cls_w2, cls_b2.reshape(1, 2))


# speedup vs baseline: 3.1354x; 1.0002x over previous
import jax
import jax.numpy as jnp
from jax import lax
from jax.experimental import pallas as pl
from jax.experimental.pallas import tpu as pltpu
from jax.experimental.pallas import tpu_sc as plsc

N = 10000
E = 160000
F = 16
ED = 4
NG = 8
NC = 2
NS = 16
NW = NC * NS
CH = 128
NCH = 40
EPW = NCH * CH
EP = NW * EPW
ACC_N = 10240
RPT = ACC_N // NS
EB = 2048

_mesh = plsc.VectorSubcoreMesh(
    core_axis_name="c", subcore_axis_name="s", num_cores=NC, num_subcores=NS)
_sc_params = pltpu.CompilerParams(use_tc_tiling_on_sc=False)


def _wid():
    return lax.axis_index("s") * NC + lax.axis_index("c")


def _gather_body(x_hbm, idx_hbm, out_hbm, idx_v, rows_v, sem):
    w = _wid()
    base = w * EPW
    pltpu.sync_copy(idx_hbm.at[w], idx_v)

    def fire(ci, carry):
        pltpu.async_copy(x_hbm.at[idx_v.at[ci]],
                         rows_v.at[pl.ds(ci * CH, CH)], sem)
        return carry

    lax.fori_loop(0, NCH, fire, 0)
    pltpu.make_async_copy(out_hbm.at[pl.ds(base, EPW)], rows_v, sem).wait()
    pltpu.sync_copy(rows_v, out_hbm.at[pl.ds(base, EPW)])


_gather = pl.kernel(
    _gather_body,
    out_type=jax.ShapeDtypeStruct((EP, F), jnp.float32),
    mesh=_mesh,
    scratch_types=[
        pltpu.VMEM((NCH, CH), jnp.int32),
        pltpu.VMEM((EPW, F), jnp.float32),
        pltpu.SemaphoreType.DMA,
    ],
    compiler_params=_sc_params,
)


def _scatter_body(m_hbm, idx_hbm, zeros_hbm, out_hbm, idx_v, rows_v, acc,
                  sem):
    cid = lax.axis_index("c")
    sid = lax.axis_index("s")
    w = _wid()
    base = w * EPW
    pltpu.sync_copy(zeros_hbm.at[pl.ds(sid * RPT, RPT)],
                    acc.at[pl.ds(sid * RPT, RPT)])
    pltpu.sync_copy(idx_hbm.at[w], idx_v)
    pltpu.sync_copy(m_hbm.at[pl.ds(base, EPW)], rows_v)
    plsc.subcore_barrier()

    def fire(ci, carry):
        pltpu.async_copy(rows_v.at[pl.ds(ci * CH, CH)],
                         acc.at[idx_v.at[ci]], sem, add=True)
        return carry

    lax.fori_loop(0, NCH, fire, 0)
    pltpu.make_async_copy(m_hbm.at[pl.ds(base, EPW)], rows_v, sem).wait()
    plsc.subcore_barrier()
    pltpu.sync_copy(acc.at[pl.ds(sid * RPT, RPT)],
                    out_hbm.at[cid, pl.ds(sid * RPT, RPT)])


_scatter = pl.kernel(
    _scatter_body,
    out_type=jax.ShapeDtypeStruct((NC, ACC_N, F), jnp.float32),
    mesh=_mesh,
    scratch_types=[
        pltpu.VMEM((NCH, CH), jnp.int32),
        pltpu.VMEM((EPW, F), jnp.float32),
        pltpu.VMEM_SHARED((ACC_N, F), jnp.float32),
        pltpu.SemaphoreType.DMA,
    ],
    compiler_params=_sc_params,
)


def _count_body(idx_hbm, zeros_hbm, onepat_hbm, out_hbm, idx_v, obuf, acc,
                sem):
    cid = lax.axis_index("c")
    sid = lax.axis_index("s")
    w = _wid()
    pltpu.sync_copy(zeros_hbm.at[pl.ds(sid * RPT, RPT)],
                    acc.at[pl.ds(sid * RPT, RPT)])
    pltpu.sync_copy(idx_hbm.at[w], idx_v)
    pltpu.sync_copy(onepat_hbm, obuf)
    plsc.subcore_barrier()

    def fire(ci, carry):
        pltpu.async_copy(obuf, acc.at[idx_v.at[ci]], sem, add=True)
        return carry

    lax.fori_loop(0, NCH, fire, 0)

    def drain(ci, carry):
        pltpu.make_async_copy(onepat_hbm, obuf, sem).wait()
        return carry

    lax.fori_loop(0, NCH, drain, 0)
    plsc.subcore_barrier()
    pltpu.sync_copy(acc.at[pl.ds(sid * RPT, RPT)],
                    out_hbm.at[cid, pl.ds(sid * RPT, RPT)])


_count = pl.kernel(
    _count_body,
    out_type=jax.ShapeDtypeStruct((NC, ACC_N, F), jnp.float32),
    mesh=_mesh,
    scratch_types=[
        pltpu.VMEM((NCH, CH), jnp.int32),
        pltpu.VMEM((CH, F), jnp.float32),
        pltpu.VMEM_SHARED((ACC_N, F), jnp.float32),
        pltpu.SemaphoreType.DMA,
    ],
    compiler_params=_sc_params,
)


def _edge_body(xj_ref, ea_ref, w1_ref, b1_ref, w2_ref, b2_ref, r_ref, s_ref,
               m_ref):
    i = pl.program_id(0)
    row = lax.broadcasted_iota(jnp.int32, (EB, F), 0) + i * EB
    xj = jnp.where(row < E, xj_ref[...], 0.0)
    h = jnp.maximum(ea_ref[...] @ w1_ref[...] + b1_ref[...], 0.0)
    we = h @ w2_ref[...] + b2_ref[...]
    x2 = xj @ r_ref[...]
    m_ref[...] = (x2 * we) @ s_ref[...]


def _edge(xj, ea, w1, b1, w2, b2, rm, sm):
    return pl.pallas_call(
        _edge_body,
        grid=(EP // EB,),
        in_specs=[
            pl.BlockSpec((EB, F), lambda i: (i, 0)),
            pl.BlockSpec((EB, ED), lambda i: (i, 0)),
            pl.BlockSpec((ED, 32), lambda i: (0, 0)),
            pl.BlockSpec((1, 32), lambda i: (0, 0)),
            pl.BlockSpec((32, 256), lambda i: (0, 0)),
            pl.BlockSpec((1, 256), lambda i: (0, 0)),
            pl.BlockSpec((F, 256), lambda i: (0, 0)),
            pl.BlockSpec((256, F), lambda i: (0, 0)),
        ],
        out_specs=pl.BlockSpec((EB, F), lambda i: (i, 0)),
        out_shape=jax.ShapeDtypeStruct((EP, F), jnp.float32),
    )(xj, ea, w1, b1, w2, b2, rm, sm)


def _fin_body(x_ref, parts_ref, cnt_ref, root_ref, bias_ref, o_ref):
    s = parts_ref[0, :N, :] + parts_ref[1, :N, :]
    c0 = cnt_ref[0, :N, 0:1] + cnt_ref[1, :N, 0:1]
    agg = s / jnp.maximum(c0, 1.0)
    o_ref[...] = jnp.maximum(
        agg + x_ref[...] @ root_ref[...] + bias_ref[...], 0.0)


def _finalize(x, parts, cnt, root, bias):
    return pl.pallas_call(
        _fin_body,
        out_shape=jax.ShapeDtypeStruct((N, F), jnp.float32),
    )(x, parts, cnt, root, bias)


def _pool_body(x_ref, b_ref, w1_ref, b1_ref, w2_ref, b2_ref, o_ref):
    xv = x_ref[...]
    oh = (b_ref[...] == lax.broadcasted_iota(jnp.int32, (1, NG), 1)
          ).astype(jnp.float32)
    ps = lax.dot_general(oh, xv, (((0,), (0,)), ((), ())))
    pc = lax.dot_general(oh, jnp.ones((N, 1), jnp.float32),
                         (((0,), (0,)), ((), ())))
    p = ps / jnp.maximum(pc, 1.0)
    h = jnp.maximum(p @ w1_ref[...] + b1_ref[...], 0.0)
    o_ref[...] = h @ w2_ref[...] + b2_ref[...]


def _pool(x, batch2d, w1, b1, w2, b2):
    return pl.pallas_call(
        _pool_body,
        out_shape=jax.ShapeDtypeStruct((NG, 2), jnp.float32),
    )(x, batch2d, w1, b1, w2, b2)


def kernel(x, edge_index, edge_attr, batch,
           en1_w1, en1_b1, en1_w2, en1_b2, root1, bias1,
           en2_w1, en2_b1, en2_w2, en2_b2, root2, bias2,
           en3_w1, en3_b1, en3_w2, en3_b2, root3, bias3,
           cls_w1, cls_b1, cls_w2, cls_b2):
    pad = EP - E
    src = jnp.concatenate(
        [edge_index[0], jnp.zeros((pad,), jnp.int32)]).reshape(NW, NCH, CH)
    dst = jnp.concatenate(
        [edge_index[1], jnp.full((pad,), N, jnp.int32)]).reshape(NW, NCH, CH)
    ea = jnp.concatenate(
        [edge_attr, jnp.zeros((pad, ED), jnp.float32)])
    zeros_acc = jnp.zeros((ACC_N, F), jnp.float32)
    onepat = jnp.zeros((CH, F), jnp.float32).at[:, 0].set(1.0)
    rm = jnp.repeat(jnp.eye(F, dtype=jnp.float32), F, axis=1)
    sm = jnp.tile(jnp.eye(F, dtype=jnp.float32), (F, 1))

    cnt = _count(dst, zeros_acc, onepat)

    xc = x
    layers = [
        (en1_w1, en1_b1, en1_w2, en1_b2, root1, bias1),
        (en2_w1, en2_b1, en2_w2, en2_b2, root2, bias2),
        (en3_w1, en3_b1, en3_w2, en3_b2, root3, bias3),
    ]
    for w1, b1, w2, b2, root, bias in layers:
        xj = _gather(xc, src)
        m = _edge(xj, ea, w1, b1.reshape(1, 32), w2, b2.reshape(1, 256),
                  rm, sm)
        parts = _scatter(m, dst, zeros_acc)
        xc = _finalize(xc, parts, cnt, root, bias.reshape(1, F))

    return _pool(xc, batch.reshape(N, 1), cls_w1, cls_b1.reshape(1, NG),
                 cls_w2, cls_b2.reshape(1, 2))


# speedup vs baseline: 3.9556x; 1.2616x over previous
import jax
import jax.numpy as jnp
from jax import lax
from jax.experimental import pallas as pl
from jax.experimental.pallas import tpu as pltpu
from jax.experimental.pallas import tpu_sc as plsc

N = 10000
E = 160000
F = 16
ED = 4
NG = 8
NC = 2
NS = 16
NW = NC * NS
CH = 128
NCH = 40
EPW = NCH * CH
EP = NW * EPW
ACC_N = 10240
RPT = ACC_N // NS
EB = 2048

_mesh = plsc.VectorSubcoreMesh(
    core_axis_name="c", subcore_axis_name="s", num_cores=NC, num_subcores=NS)
_sc_params = pltpu.CompilerParams(use_tc_tiling_on_sc=False)


def _wid():
    return lax.axis_index("s") * NC + lax.axis_index("c")


def _gather_body(x_hbm, idx_hbm, out_hbm, idx_v, rows_v, sem):
    w = _wid()
    base = w * EPW
    pltpu.sync_copy(idx_hbm.at[w], idx_v)

    def fire(ci, carry):
        pltpu.async_copy(x_hbm.at[idx_v.at[ci]],
                         rows_v.at[pl.ds(ci * CH, CH)], sem)
        return carry

    lax.fori_loop(0, NCH, fire, 0)
    pltpu.make_async_copy(out_hbm.at[pl.ds(base, EPW)], rows_v, sem).wait()
    pltpu.sync_copy(rows_v, out_hbm.at[pl.ds(base, EPW)])


_gather = pl.kernel(
    _gather_body,
    out_type=jax.ShapeDtypeStruct((EP, F), jnp.float32),
    mesh=_mesh,
    scratch_types=[
        pltpu.VMEM((NCH, CH), jnp.int32),
        pltpu.VMEM((EPW, F), jnp.float32),
        pltpu.SemaphoreType.DMA,
    ],
    compiler_params=_sc_params,
)


def _scatter_body(m_hbm, idx_hbm, zeros_hbm, out_hbm, idx_v, rows_v, acc,
                  sem):
    cid = lax.axis_index("c")
    sid = lax.axis_index("s")
    w = _wid()
    pltpu.sync_copy(zeros_hbm.at[pl.ds(sid * RPT, RPT)],
                    acc.at[pl.ds(sid * RPT, RPT)])
    pltpu.sync_copy(idx_hbm.at[w], idx_v)
    base = w * EPW
    pltpu.sync_copy(m_hbm.at[pl.ds(base, EPW)], rows_v)
    plsc.subcore_barrier()

    def fire(ci, carry):
        pltpu.async_copy(rows_v.at[pl.ds(ci * CH, CH)],
                         acc.at[idx_v.at[ci]], sem, add=True)
        return carry

    lax.fori_loop(0, NCH, fire, 0)
    pltpu.make_async_copy(m_hbm.at[pl.ds(base, EPW)], rows_v, sem).wait()
    plsc.subcore_barrier()
    pltpu.sync_copy(acc.at[pl.ds(sid * RPT, RPT)],
                    out_hbm.at[cid, pl.ds(sid * RPT, RPT)])


_scatter = pl.kernel(
    _scatter_body,
    out_type=jax.ShapeDtypeStruct((NC, ACC_N, F), jnp.float32),
    mesh=_mesh,
    scratch_types=[
        pltpu.VMEM((NCH, CH), jnp.int32),
        pltpu.VMEM((EPW, F), jnp.float32),
        pltpu.VMEM_SHARED((ACC_N, F), jnp.float32),
        pltpu.SemaphoreType.DMA,
    ],
    compiler_params=_sc_params,
)


def _count_body(idx_hbm, zeros_hbm, onepat_hbm, out_hbm, idx_v, obuf, acc,
                sem):
    cid = lax.axis_index("c")
    sid = lax.axis_index("s")
    w = _wid()
    pltpu.sync_copy(zeros_hbm.at[pl.ds(sid * RPT, RPT)],
                    acc.at[pl.ds(sid * RPT, RPT)])
    pltpu.sync_copy(idx_hbm.at[w], idx_v)
    pltpu.sync_copy(onepat_hbm, obuf)
    plsc.subcore_barrier()

    def fire(ci, carry):
        pltpu.async_copy(obuf, acc.at[idx_v.at[ci]], sem, add=True)
        return carry

    lax.fori_loop(0, NCH, fire, 0)

    def drain(ci, carry):
        pltpu.make_async_copy(onepat_hbm, obuf, sem).wait()
        return carry

    lax.fori_loop(0, NCH, drain, 0)
    plsc.subcore_barrier()
    pltpu.sync_copy(acc.at[pl.ds(sid * RPT, RPT)],
                    out_hbm.at[cid, pl.ds(sid * RPT, RPT)])


_count = pl.kernel(
    _count_body,
    out_type=jax.ShapeDtypeStruct((NC, ACC_N, F), jnp.float32),
    mesh=_mesh,
    scratch_types=[
        pltpu.VMEM((NCH, CH), jnp.int32),
        pltpu.VMEM((CH, F), jnp.float32),
        pltpu.VMEM_SHARED((ACC_N, F), jnp.float32),
        pltpu.SemaphoreType.DMA,
    ],
    compiler_params=_sc_params,
)


def _edge_body(xj_ref, ea_ref, bdw1_ref, b1t_ref, bdw2_ref, b2t_ref, bdr_ref,
               bds_ref, m_ref):
    i = pl.program_id(0)
    prow = (lax.broadcasted_iota(jnp.int32, (EB // 8, 128), 0)
            + i * (EB // 8))
    xjp = jnp.where(prow < E // 8, xj_ref[...], 0.0)
    hp32 = jnp.maximum(ea_ref[...] @ bdw1_ref[...] + b1t_ref[...], 0.0)
    hp4 = jnp.reshape(hp32, (EB // 4, 128))
    wep4 = hp4 @ bdw2_ref[...] + b2t_ref[...]
    wep8 = jnp.reshape(wep4, (EB // 8, 2048))
    x2p8 = xjp @ bdr_ref[...]
    m_ref[...] = (x2p8 * wep8) @ bds_ref[...]


def _edge(xj, ea, bdw1, b1t, bdw2, b2t, bdr, bds):
    return pl.pallas_call(
        _edge_body,
        grid=(EP // EB,),
        in_specs=[
            pl.BlockSpec((EB // 8, 128), lambda i: (i, 0)),
            pl.BlockSpec((EB * ED // 128, 128), lambda i: (i, 0)),
            pl.BlockSpec((128, 1024), lambda i: (0, 0)),
            pl.BlockSpec((1, 1024), lambda i: (0, 0)),
            pl.BlockSpec((128, 1024), lambda i: (0, 0)),
            pl.BlockSpec((1, 1024), lambda i: (0, 0)),
            pl.BlockSpec((128, 2048), lambda i: (0, 0)),
            pl.BlockSpec((2048, 128), lambda i: (0, 0)),
        ],
        out_specs=pl.BlockSpec((EB // 8, 128), lambda i: (i, 0)),
        out_shape=jax.ShapeDtypeStruct((EP * F // 128, 128), jnp.float32),
    )(xj, ea, bdw1, b1t, bdw2, b2t, bdr, bds)


def _fin_body(x_ref, parts_ref, cnt_ref, root_ref, bias_ref, o_ref):
    s = parts_ref[0, :N, :] + parts_ref[1, :N, :]
    c0 = cnt_ref[0, :N, 0:1] + cnt_ref[1, :N, 0:1]
    agg = s / jnp.maximum(c0, 1.0)
    o_ref[...] = jnp.maximum(
        agg + x_ref[...] @ root_ref[...] + bias_ref[...], 0.0)


def _finalize(x, parts, cnt, root, bias):
    return pl.pallas_call(
        _fin_body,
        out_shape=jax.ShapeDtypeStruct((N, F), jnp.float32),
    )(x, parts, cnt, root, bias)


def _pool_body(x_ref, b_ref, w1_ref, b1_ref, w2_ref, b2_ref, o_ref):
    xv = x_ref[...]
    oh = (b_ref[...] == lax.broadcasted_iota(jnp.int32, (1, NG), 1)
          ).astype(jnp.float32)
    ps = lax.dot_general(oh, xv, (((0,), (0,)), ((), ())))
    pc = lax.dot_general(oh, jnp.ones((N, 1), jnp.float32),
                         (((0,), (0,)), ((), ())))
    p = ps / jnp.maximum(pc, 1.0)
    h = jnp.maximum(p @ w1_ref[...] + b1_ref[...], 0.0)
    o_ref[...] = h @ w2_ref[...] + b2_ref[...]


def _pool(x, batch2d, w1, b1, w2, b2):
    return pl.pallas_call(
        _pool_body,
        out_shape=jax.ShapeDtypeStruct((NG, 2), jnp.float32),
    )(x, batch2d, w1, b1, w2, b2)


def kernel(x, edge_index, edge_attr, batch,
           en1_w1, en1_b1, en1_w2, en1_b2, root1, bias1,
           en2_w1, en2_b1, en2_w2, en2_b2, root2, bias2,
           en3_w1, en3_b1, en3_w2, en3_b2, root3, bias3,
           cls_w1, cls_b1, cls_w2, cls_b2):
    pad = EP - E
    src = jnp.concatenate(
        [edge_index[0], jnp.zeros((pad,), jnp.int32)]).reshape(NW, NCH, CH)
    dst = jnp.concatenate(
        [edge_index[1], jnp.full((pad,), N, jnp.int32)]).reshape(NW, NCH, CH)
    ea = jnp.concatenate(
        [edge_attr, jnp.zeros((pad, ED), jnp.float32)]
    ).reshape(EP * ED // 128, 128)
    zeros_acc = jnp.zeros((ACC_N, F), jnp.float32)
    onepat = jnp.zeros((CH, F), jnp.float32).at[:, 0].set(1.0)
    eye8 = jnp.eye(8, dtype=jnp.float32)
    rm = jnp.repeat(jnp.eye(F, dtype=jnp.float32), F, axis=1)
    sm = jnp.tile(jnp.eye(F, dtype=jnp.float32), (F, 1))
    bdr = jnp.kron(eye8, rm)
    bds = jnp.kron(eye8, sm)

    cnt = _count(dst, zeros_acc, onepat)

    xc = x
    layers = [
        (en1_w1, en1_b1, en1_w2, en1_b2, root1, bias1),
        (en2_w1, en2_b1, en2_w2, en2_b2, root2, bias2),
        (en3_w1, en3_b1, en3_w2, en3_b2, root3, bias3),
    ]
    for w1, b1, w2, b2, root, bias in layers:
        bdw1 = jnp.kron(jnp.eye(32, dtype=jnp.float32), w1)
        bdw2 = jnp.kron(jnp.eye(ED, dtype=jnp.float32), w2)
        b1t = jnp.tile(b1, 32).reshape(1, 1024)
        b2t = jnp.tile(b2, ED).reshape(1, 1024)
        xj = _gather(xc, src).reshape(EP * F // 128, 128)
        m = _edge(xj, ea, bdw1, b1t, bdw2, b2t, bdr, bds)
        parts = _scatter(m.reshape(EP, F), dst, zeros_acc)
        xc = _finalize(xc, parts, cnt, root, bias.reshape(1, F))

    return _pool(xc, batch.reshape(N, 1), cls_w1, cls_b1.reshape(1, NG),
                 cls_w2, cls_b2.reshape(1, 2))
